# baseline (speedup 1.0000x reference)
import jax
import jax.numpy as jnp
import numpy as np
from jax import lax
from jax.experimental import pallas as pl
from jax.experimental.pallas import tpu as pltpu
from jax.experimental.pallas import tpu_sc as plsc

BATCH = 16384
EMB = 64
PACK = 128
NC = 2
NS = 16
NW = NC * NS
BPW = BATCH // NW
CHUNK = 16
NQ = 4
QH = BPW // NQ
QCHUNKS = QH // CHUNK

HI_MASK = np.int32(-65536)


def _mf_body(u_id_hbm, i_id_hbm, user_emb_hbm, user_bias_hbm,
             item_emb_hbm, item_bias_hbm, mean_hbm, out_hbm,
             u_idx_v, i_idx_v, u_rows0, i_rows0, u_rows1, i_rows1,
             u_bias_v, i_bias_v, mean_v, out_v, sem0, sem1, bsem):
    wid = lax.axis_index("s") * NC + lax.axis_index("c")
    base = wid * BPW

    pltpu.sync_copy(u_id_hbm.at[pl.ds(base, BPW)], u_idx_v)
    pltpu.sync_copy(i_id_hbm.at[pl.ds(base, BPW)], i_idx_v)
    pltpu.sync_copy(mean_hbm, mean_v)

    ub_cp = pltpu.async_copy(user_bias_hbm.at[u_idx_v], u_bias_v, bsem)
    ib_cp = pltpu.async_copy(item_bias_hbm.at[i_idx_v], i_bias_v, bsem)

    bufs = ((u_rows0, i_rows0, sem0), (u_rows1, i_rows1, sem1))
    lanes = lax.iota(jnp.int32, 16)

    def fetch_quarter(q):
        u_buf, i_buf, sem = bufs[q % 2]
        q0 = q * QH

        def fetch(j, carry):
            uv = u_idx_v[pl.ds(q0 + j * CHUNK, CHUNK)]
            iv = i_idx_v[pl.ds(q0 + j * CHUNK, CHUNK)]
            for l in range(CHUNK):
                b = j * CHUNK + l
                pltpu.async_copy(
                    user_emb_hbm.at[pl.ds(lax.shift_right_logical(uv[l], 2), 1), :],
                    u_buf.at[pl.ds(b, 1), :], sem)
                pltpu.async_copy(
                    item_emb_hbm.at[pl.ds(lax.shift_right_logical(iv[l], 2), 1), :],
                    i_buf.at[pl.ds(b, 1), :], sem)
            return carry

        lax.fori_loop(0, QCHUNKS, fetch, 0)

    def drain_quarter(q):
        u_buf, i_buf, sem = bufs[q % 2]
        pltpu.make_async_copy(user_emb_hbm.at[pl.ds(0, QH), :],
                              u_buf, sem).wait()
        pltpu.make_async_copy(item_emb_hbm.at[pl.ds(0, QH), :],
                              i_buf, sem).wait()

    def dot16(buf, b, col0):
        w0 = plsc.bitcast(buf[b, pl.ds(col0, 16)], jnp.int32)
        w1 = plsc.bitcast(buf[b, pl.ds(col0 + 16, 16)], jnp.int32)
        lo0 = plsc.bitcast(lax.shift_left(w0, 16), jnp.float32)
        hi0 = plsc.bitcast(lax.bitwise_and(w0, HI_MASK), jnp.float32)
        lo1 = plsc.bitcast(lax.shift_left(w1, 16), jnp.float32)
        hi1 = plsc.bitcast(lax.bitwise_and(w1, HI_MASK), jnp.float32)
        return lo0, hi0, lo1, hi1

    def compute_quarter(q):
        u_buf, i_buf, _ = bufs[q % 2]
        q0 = q * QH
        mean_vec = mean_v[pl.ds(0, 16)]

        def body(j, carry):
            b0 = j * CHUNK
            uv = u_idx_v[pl.ds(q0 + b0, CHUNK)]
            iv = i_idx_v[pl.ds(q0 + b0, CHUNK)]
            acc = jnp.zeros((16,), jnp.float32)
            for l in range(CHUNK):
                b = b0 + l
                cu = lax.shift_left(lax.bitwise_and(uv[l], 3), 5)
                ci = lax.shift_left(lax.bitwise_and(iv[l], 3), 5)
                ulo0, uhi0, ulo1, uhi1 = dot16(u_buf, b, cu)
                ilo0, ihi0, ilo1, ihi1 = dot16(i_buf, b, ci)
                p = ulo0 * ilo0 + uhi0 * ihi0 + ulo1 * ilo1 + uhi1 * ihi1
                s = jnp.sum(p)
                acc = jnp.where(lanes == l, s, acc)
            ub = u_bias_v[pl.ds(q0 + b0, CHUNK)]
            ib = i_bias_v[pl.ds(q0 + b0, CHUNK)]
            out_v[pl.ds(q0 + b0, CHUNK)] = acc + ub + ib + mean_vec
            return carry

        lax.fori_loop(0, QCHUNKS, body, 0)

    fetch_quarter(0)
    fetch_quarter(1)
    drain_quarter(0)
    compute_quarter(0)
    fetch_quarter(2)
    drain_quarter(1)
    compute_quarter(1)
    fetch_quarter(3)
    drain_quarter(2)
    compute_quarter(2)
    drain_quarter(3)
    compute_quarter(3)

    ub_cp.wait()
    ib_cp.wait()
    pltpu.sync_copy(out_v, out_hbm.at[pl.ds(base, BPW)])


_mf = pl.kernel(
    _mf_body,
    out_type=jax.ShapeDtypeStruct((BATCH,), jnp.float32),
    mesh=plsc.VectorSubcoreMesh(core_axis_name="c", subcore_axis_name="s"),
    compiler_params=pltpu.CompilerParams(needs_layout_passes=False),
    scratch_types=[
        pltpu.VMEM((BPW,), jnp.int32),
        pltpu.VMEM((BPW,), jnp.int32),
        pltpu.VMEM((QH, PACK), jnp.float32),
        pltpu.VMEM((QH, PACK), jnp.float32),
        pltpu.VMEM((QH, PACK), jnp.float32),
        pltpu.VMEM((QH, PACK), jnp.float32),
        pltpu.VMEM((BPW,), jnp.float32),
        pltpu.VMEM((BPW,), jnp.float32),
        pltpu.VMEM((16,), jnp.float32),
        pltpu.VMEM((BPW,), jnp.float32),
        pltpu.SemaphoreType.DMA,
        pltpu.SemaphoreType.DMA,
        pltpu.SemaphoreType.DMA,
    ],
)


def _pack_rows(emb):
    n = emb.shape[0]
    pairs = emb.astype(jnp.bfloat16).reshape(n, EMB // 2, 2)
    packed = jax.lax.bitcast_convert_type(pairs, jnp.float32)
    return packed.reshape(n // 4, PACK)


def kernel(u_id, i_id, user_emb, user_bias, item_emb, item_bias, mean):
    return _mf(u_id.astype(jnp.int32), i_id.astype(jnp.int32),
               _pack_rows(user_emb), user_bias.reshape(-1),
               _pack_rows(item_emb), item_bias.reshape(-1),
               jnp.broadcast_to(mean, (16,)))
